# baseline (speedup 1.0000x reference)
import functools

import jax
import jax.numpy as jnp
from jax import lax
from jax.experimental import pallas as pl
from jax.experimental.pallas import tpu as pltpu
from jax.experimental.pallas import tpu_sc as plsc

INPUT_BITS = 16384
OUTPUT_BITS = 65536
N_BITS = 8
N_CELLS = 1 << N_BITS

NC = 2
NS = 16
L = 16
NW = NC * NS
N_PER_W = OUTPUT_BITS // NW
BLOCKS = N_PER_W // 128
GROUPS = 128 // L


def _build():
    mesh = plsc.VectorSubcoreMesh(core_axis_name="c", subcore_axis_name="s")

    @functools.partial(
        pl.kernel,
        mesh=mesh,
        compiler_params=pltpu.CompilerParams(needs_layout_passes=False),
        out_type=jax.ShapeDtypeStruct((OUTPUT_BITS,), jnp.float32),
        scratch_types=[
            pltpu.VMEM((INPUT_BITS,), jnp.int32),
            pltpu.VMEM((N_PER_W * N_BITS,), jnp.int32),
            pltpu.VMEM((BLOCKS, 128), jnp.int32),
            pltpu.VMEM((N_PER_W,), jnp.float32),
            pltpu.SemaphoreType.DMA,
            pltpu.SemaphoreType.DMA,
            pltpu.SemaphoreType.DMA,
        ],
    )
    def ram_lookup(x_hbm, map_hbm, mem_hbm, out_hbm, x_v, map_v, idx_v, res_v,
                   in_sem, m_sem, g_sem):
        wid = lax.axis_index("s") * NC + lax.axis_index("c")
        base = wid * N_PER_W

        chunks = 4
        mb_per_chunk = N_PER_W * N_BITS // chunks
        blk_per_chunk = BLOCKS // chunks
        grp_per_chunk = BLOCKS * GROUPS // chunks

        cp_x = pltpu.async_copy(x_hbm, x_v, in_sem)
        map_cps = [
            pltpu.async_copy(
                map_hbm.at[pl.ds(base * N_BITS + c * mb_per_chunk,
                                 mb_per_chunk)],
                map_v.at[pl.ds(c * mb_per_chunk, mb_per_chunk)], m_sem)
            for c in range(chunks)
        ]
        cp_x.wait()

        lanes = lax.iota(jnp.int32, L)
        lane_part = ((lanes >> 3) << 11) | ((lanes & 7) << 7)

        gathers = []
        for c in range(chunks):
            map_cps[c].wait()

            @plsc.parallel_loop(c * grp_per_chunk, (c + 1) * grp_per_chunk,
                                unroll=2)
            def body(g):
                addr = jnp.zeros((L,), jnp.int32)
                mb = (g >> 3) * 1024 + (g & 7) * L
                for j in range(N_BITS):
                    mj = map_v[pl.ds(mb + j * 128, L)]
                    bit = plsc.load_gather(x_v, [mj])
                    addr = addr | (bit << j)
                n0 = base + g * L
                flat = (((n0 >> 3) << 11) + lane_part
                        + ((addr >> 7) << 10) + (addr & 127))
                idx_v[g >> 3, pl.ds((g & 7) * L, L)] = flat

            for t in range(c * blk_per_chunk, (c + 1) * blk_per_chunk):
                gathers.append(
                    pltpu.async_copy(mem_hbm.at[idx_v.at[t]],
                                     res_v.at[pl.ds(t * 128, 128)], g_sem))
        for cp in gathers:
            cp.wait()
        pltpu.sync_copy(res_v, out_hbm.at[pl.ds(base, N_PER_W)])

    return ram_lookup


_RAM_LOOKUP = _build()


@jax.jit
def kernel(x, mapping, memory):
    mem_phys = memory.reshape(8192, 8, 2, 128).transpose(0, 2, 1, 3).reshape(-1)
    map_phys = mapping.T.reshape(N_BITS, 512, 128).transpose(1, 0, 2).reshape(-1)
    return _RAM_LOOKUP(x, map_phys, mem_phys)


# speedup vs baseline: 1.0722x; 1.0722x over previous
import functools

import jax
import jax.numpy as jnp
from jax import lax
from jax.experimental import pallas as pl
from jax.experimental.pallas import tpu as pltpu
from jax.experimental.pallas import tpu_sc as plsc

INPUT_BITS = 16384
OUTPUT_BITS = 65536
N_BITS = 8
N_CELLS = 1 << N_BITS

NC = 2
NS = 16
L = 16
NW = NC * NS
N_PER_W = OUTPUT_BITS // NW
BLOCKS = N_PER_W // 128
GROUPS = 128 // L


def _build():
    mesh = plsc.VectorSubcoreMesh(core_axis_name="c", subcore_axis_name="s")

    @functools.partial(
        pl.kernel,
        mesh=mesh,
        compiler_params=pltpu.CompilerParams(needs_layout_passes=False),
        out_type=jax.ShapeDtypeStruct((OUTPUT_BITS,), jnp.float32),
        scratch_types=[
            pltpu.VMEM((INPUT_BITS,), jnp.int32),
            pltpu.VMEM_SHARED((INPUT_BITS,), jnp.int32),
            pltpu.VMEM((N_PER_W * N_BITS,), jnp.int32),
            pltpu.VMEM((BLOCKS, 128), jnp.int32),
            pltpu.VMEM((N_PER_W,), jnp.float32),
            pltpu.SemaphoreType.DMA,
            pltpu.SemaphoreType.DMA,
            pltpu.SemaphoreType.DMA,
        ],
    )
    def ram_lookup(x_hbm, map_hbm, mem_hbm, out_hbm, x_v, x_sh, map_v, idx_v, res_v,
                   in_sem, m_sem, g_sem):
        wid = lax.axis_index("s") * NC + lax.axis_index("c")
        base = wid * N_PER_W

        chunks = 4
        mb_per_chunk = N_PER_W * N_BITS // chunks
        blk_per_chunk = BLOCKS // chunks
        grp_per_chunk = BLOCKS * GROUPS // chunks

        map_cps = [
            pltpu.async_copy(
                map_hbm.at[pl.ds(base * N_BITS + c * mb_per_chunk,
                                 mb_per_chunk)],
                map_v.at[pl.ds(c * mb_per_chunk, mb_per_chunk)], m_sem)
            for c in range(chunks)
        ]

        @pl.when(lax.axis_index("s") == 0)
        def _():
            pltpu.sync_copy(x_hbm, x_sh)

        plsc.subcore_barrier()
        pltpu.async_copy(x_sh, x_v, in_sem).wait()

        lanes = lax.iota(jnp.int32, L)
        lane_part = ((lanes >> 3) << 11) | ((lanes & 7) << 7)

        gathers = []
        for c in range(chunks):
            map_cps[c].wait()

            @plsc.parallel_loop(c * grp_per_chunk, (c + 1) * grp_per_chunk,
                                unroll=2)
            def body(g):
                addr = jnp.zeros((L,), jnp.int32)
                mb = (g >> 3) * 1024 + (g & 7) * L
                for j in range(N_BITS):
                    mj = map_v[pl.ds(mb + j * 128, L)]
                    bit = plsc.load_gather(x_v, [mj])
                    addr = addr | (bit << j)
                n0 = base + g * L
                flat = (((n0 >> 3) << 11) + lane_part
                        + ((addr >> 7) << 10) + (addr & 127))
                idx_v[g >> 3, pl.ds((g & 7) * L, L)] = flat

            for t in range(c * blk_per_chunk, (c + 1) * blk_per_chunk):
                gathers.append(
                    pltpu.async_copy(mem_hbm.at[idx_v.at[t]],
                                     res_v.at[pl.ds(t * 128, 128)], g_sem))
        for cp in gathers:
            cp.wait()
        pltpu.sync_copy(res_v, out_hbm.at[pl.ds(base, N_PER_W)])

    return ram_lookup


_RAM_LOOKUP = _build()


@jax.jit
def kernel(x, mapping, memory):
    mem_phys = memory.reshape(8192, 8, 2, 128).transpose(0, 2, 1, 3).reshape(-1)
    map_phys = mapping.T.reshape(N_BITS, 512, 128).transpose(1, 0, 2).reshape(-1)
    return _RAM_LOOKUP(x, map_phys, mem_phys)
